# Initial kernel scaffold; baseline (speedup 1.0000x reference)
import jax
import jax.numpy as jnp
from jax.experimental import pallas as pl


def kernel(y0, base_idx, slab):
    raise NotImplementedError("write your pallas kernel here")


# speedup vs baseline: 2.5888x; 2.5888x over previous
import jax
import jax.numpy as jnp
from jax.experimental import pallas as pl
from jax.experimental.pallas import tpu as pltpu

T, DY, H = 8192, 256, 1024
R_W1Y, R_W2, R_B2, R_W3, R_B3 = 8192, 8448, 9472, 9480, 10504
DT = 100.0
SLOTS = (0, 0, 0, 0, 1, 1, 1, 1, 1, 1, 1, 1, 1, 2, 2, 2)
NUM_OFF = 3
TILE_B = 128
N_STEPS = 16


def _euler_kernel(idx_ref, y0_ref, slab_ref, out_ref):
    y = y0_ref[...]

    TC = 2048
    col = jax.lax.broadcasted_iota(jnp.int32, (TILE_B, TC), 1)
    idx0 = idx_ref[...]
    hx = []
    for o in range(NUM_OFF):
        idx = jnp.clip(idx0 + o, 0, T - 1)
        acc = jnp.zeros((TILE_B, H), jnp.float32)
        for c in range(T // TC):
            onehot = (col == (idx - c * TC)).astype(jnp.float32)
            acc = acc + jnp.dot(onehot, slab_ref[c * TC:(c + 1) * TC, :],
                                preferred_element_type=jnp.float32)
        hx.append(acc)

    w1y = slab_ref[R_W1Y:R_W1Y + DY, :]
    w2 = slab_ref[R_W2:R_W2 + H, :]
    b2 = slab_ref[R_B2:R_B2 + 1, :]
    w3 = slab_ref[R_W3:R_W3 + H, :DY]
    b3 = slab_ref[R_B3:R_B3 + 1, :DY]

    for k in range(N_STEPS):
        h1 = jnp.tanh(hx[SLOTS[k]]
                      + jnp.dot(y, w1y, preferred_element_type=jnp.float32))
        h2 = jnp.tanh(jnp.dot(h1, w2, preferred_element_type=jnp.float32) + b2)
        y = y + DT * (jnp.dot(h2, w3, preferred_element_type=jnp.float32) + b3)

    out_ref[...] = y


def kernel(y0, base_idx, slab):
    batch, dy = y0.shape
    assert dy == DY
    idx_in = base_idx.astype(jnp.int32).reshape(batch, 1)

    out = pl.pallas_call(
        _euler_kernel,
        out_shape=jax.ShapeDtypeStruct((batch, DY), jnp.float32),
        grid=(batch // TILE_B,),
        in_specs=[
            pl.BlockSpec((TILE_B, 1), lambda i: (i, 0)),
            pl.BlockSpec((TILE_B, DY), lambda i: (i, 0)),
            pl.BlockSpec(slab.shape, lambda i: (0, 0)),
        ],
        out_specs=pl.BlockSpec((TILE_B, DY), lambda i: (i, 0)),
        compiler_params=pltpu.CompilerParams(
            dimension_semantics=("parallel",)),
    )(idx_in, y0, slab)
    return out


# speedup vs baseline: 2.8773x; 1.1114x over previous
import jax
import jax.numpy as jnp
from jax.experimental import pallas as pl
from jax.experimental.pallas import tpu as pltpu

T, DY, H = 8192, 256, 1024
R_W1Y, R_W2, R_B2, R_W3, R_B3 = 8192, 8448, 9472, 9480, 10504
DT = 100.0
SLOTS = (0, 0, 0, 0, 1, 1, 1, 1, 1, 1, 1, 1, 1, 2, 2, 2)
NUM_OFF = 3
TILE_B = 128
N_STEPS = 16
P = 8
S = TILE_B + 1

W_W1Y = 0
W_W2 = R_W2 - R_W1Y
W_B2 = R_B2 - R_W1Y
W_W3 = R_W3 - R_W1Y
W_B3 = R_B3 - R_W1Y
W_ROWS = R_B3 + 8 - R_W1Y


def _euler_kernel(idx_sref, y0_ref, xw_ref, w_ref, out_ref, g0, g1, g2):
    i = pl.program_id(0)
    g = (g0, g1, g2)

    for mi in range(TILE_B):
        b = idx_sref[i * TILE_B + mi]
        for o in range(NUM_OFF):
            r = jnp.minimum(b + o, T - 1) if o else b
            src = pl.multiple_of(r * P, P)
            g[o][mi:mi + P * S:S, :] = xw_ref[pl.ds(src, P), :]

    hx = [jnp.concatenate([go[pl.ds(j * S, TILE_B), :] for j in range(P)],
                          axis=-1)
          for go in g]

    w1y = w_ref[W_W1Y:W_W1Y + DY, :]
    w2 = w_ref[W_W2:W_W2 + H, :]
    b2 = w_ref[W_B2:W_B2 + 1, :]
    w3 = w_ref[W_W3:W_W3 + H, :DY]
    b3 = w_ref[W_B3:W_B3 + 1, :DY]

    y = y0_ref[...]
    for k in range(N_STEPS):
        h1 = jnp.tanh(hx[SLOTS[k]]
                      + jnp.dot(y, w1y, preferred_element_type=jnp.float32))
        h2 = jnp.tanh(jnp.dot(h1, w2, preferred_element_type=jnp.float32) + b2)
        y = y + DT * (jnp.dot(h2, w3, preferred_element_type=jnp.float32) + b3)

    out_ref[...] = y


def kernel(y0, base_idx, slab):
    batch, dy = y0.shape
    assert dy == DY
    idx = base_idx.astype(jnp.int32)
    xw = slab[:T].reshape(T * P, 128)
    w = slab[R_W1Y:R_W1Y + W_ROWS]

    out = pl.pallas_call(
        _euler_kernel,
        out_shape=jax.ShapeDtypeStruct((batch, DY), jnp.float32),
        grid_spec=pltpu.PrefetchScalarGridSpec(
            num_scalar_prefetch=1,
            grid=(batch // TILE_B,),
            in_specs=[
                pl.BlockSpec((TILE_B, DY), lambda i, idxs: (i, 0)),
                pl.BlockSpec(xw.shape, lambda i, idxs: (0, 0)),
                pl.BlockSpec(w.shape, lambda i, idxs: (0, 0)),
            ],
            out_specs=pl.BlockSpec((TILE_B, DY), lambda i, idxs: (i, 0)),
            scratch_shapes=[pltpu.VMEM((P * S, 128), jnp.float32)
                            for _ in range(NUM_OFF)],
        ),
        compiler_params=pltpu.CompilerParams(
            dimension_semantics=("parallel",)),
    )(idx, y0, xw, w)
    return out


# speedup vs baseline: 3.5213x; 1.2238x over previous
import jax
import jax.numpy as jnp
from jax.experimental import pallas as pl
from jax.experimental.pallas import tpu as pltpu

T, DY, H = 8192, 256, 1024
R_W1Y, R_W2, R_B2, R_W3, R_B3 = 8192, 8448, 9472, 9480, 10504
DT = 100.0
SLOTS = (0, 0, 0, 0, 1, 1, 1, 1, 1, 1, 1, 1, 1, 2, 2, 2)
NUM_OFF = 3
TILE_B = 128
N_STEPS = 16


def _euler_kernel(idx_sref, y0_ref, slab_ref, out_ref, g0, g1, g2):
    i = pl.program_id(0)
    g = (g0, g1, g2)

    for mi in range(TILE_B):
        b = idx_sref[i * TILE_B + mi]
        for o in range(NUM_OFF):
            r = jnp.minimum(b + o, T - 1) if o else b
            c8 = pl.multiple_of((r >> 3) << 3, 8)
            chunk = slab_ref[pl.ds(c8, 8), :]
            row = pltpu.roll(chunk, -(r & 7), axis=0)[0:1, :]
            g[o][mi:mi + 1, :] = row

    hx = [go[...] for go in g]

    w1y = slab_ref[R_W1Y:R_W1Y + DY, :]
    w2 = slab_ref[R_W2:R_W2 + H, :]
    b2 = slab_ref[R_B2:R_B2 + 1, :]
    w3 = slab_ref[R_W3:R_W3 + H, :DY]
    b3 = slab_ref[R_B3:R_B3 + 1, :DY]

    y = y0_ref[...]
    for k in range(N_STEPS):
        h1 = jnp.tanh(hx[SLOTS[k]]
                      + jnp.dot(y, w1y, preferred_element_type=jnp.float32))
        h2 = jnp.tanh(jnp.dot(h1, w2, preferred_element_type=jnp.float32) + b2)
        y = y + DT * (jnp.dot(h2, w3, preferred_element_type=jnp.float32) + b3)

    out_ref[...] = y


def kernel(y0, base_idx, slab):
    batch, dy = y0.shape
    assert dy == DY
    idx = base_idx.astype(jnp.int32)

    out = pl.pallas_call(
        _euler_kernel,
        out_shape=jax.ShapeDtypeStruct((batch, DY), jnp.float32),
        grid_spec=pltpu.PrefetchScalarGridSpec(
            num_scalar_prefetch=1,
            grid=(batch // TILE_B,),
            in_specs=[
                pl.BlockSpec((TILE_B, DY), lambda i, idxs: (i, 0)),
                pl.BlockSpec(slab.shape, lambda i, idxs: (0, 0)),
            ],
            out_specs=pl.BlockSpec((TILE_B, DY), lambda i, idxs: (i, 0)),
            scratch_shapes=[pltpu.VMEM((TILE_B, H), jnp.float32)
                            for _ in range(NUM_OFF)],
        ),
        compiler_params=pltpu.CompilerParams(
            dimension_semantics=("parallel",)),
    )(idx, y0, slab)
    return out


# speedup vs baseline: 4.7389x; 1.3458x over previous
import jax
import jax.numpy as jnp
from jax.experimental import pallas as pl
from jax.experimental.pallas import tpu as pltpu

T, DY, H = 8192, 256, 1024
R_W1Y, R_W2, R_B2, R_W3, R_B3 = 8192, 8448, 9472, 9480, 10504
DT = 100.0
SLOTS = (0, 0, 0, 0, 1, 1, 1, 1, 1, 1, 1, 1, 1, 2, 2, 2)
NUM_OFF = 3
TILE_B = 256
N_STEPS = 16


def _euler_kernel(idx_sref, y0_ref, slab_ref, out_ref, g0, g1, g2):
    i = pl.program_id(0)
    g = (g0, g1, g2)

    for mi in range(TILE_B):
        b = idx_sref[i * TILE_B + mi]
        for o in range(NUM_OFF):
            r = jnp.minimum(b + o, T - 1) if o else b
            c8 = pl.multiple_of((r >> 3) << 3, 8)
            chunk = slab_ref[pl.ds(c8, 8), :]
            row = pltpu.roll(chunk, -(r & 7), axis=0)[0:1, :]
            g[o][mi:mi + 1, :] = row

    hx = [go[...] for go in g]

    w1y = slab_ref[R_W1Y:R_W1Y + DY, :]
    w2 = slab_ref[R_W2:R_W2 + H, :]
    b2 = slab_ref[R_B2:R_B2 + 1, :]
    w3 = slab_ref[R_W3:R_W3 + H, :DY]
    b3 = slab_ref[R_B3:R_B3 + 1, :DY]

    y = y0_ref[...]
    for k in range(N_STEPS):
        h1 = jnp.tanh(hx[SLOTS[k]]
                      + jnp.dot(y, w1y, preferred_element_type=jnp.float32))
        h2 = jnp.tanh(jnp.dot(h1, w2, preferred_element_type=jnp.float32) + b2)
        y = y + DT * (jnp.dot(h2, w3, preferred_element_type=jnp.float32) + b3)

    out_ref[...] = y


def kernel(y0, base_idx, slab):
    batch, dy = y0.shape
    assert dy == DY
    idx = base_idx.astype(jnp.int32)

    out = pl.pallas_call(
        _euler_kernel,
        out_shape=jax.ShapeDtypeStruct((batch, DY), jnp.float32),
        grid_spec=pltpu.PrefetchScalarGridSpec(
            num_scalar_prefetch=1,
            grid=(batch // TILE_B,),
            in_specs=[
                pl.BlockSpec((TILE_B, DY), lambda i, idxs: (i, 0)),
                pl.BlockSpec(slab.shape, lambda i, idxs: (0, 0)),
            ],
            out_specs=pl.BlockSpec((TILE_B, DY), lambda i, idxs: (i, 0)),
            scratch_shapes=[pltpu.VMEM((TILE_B, H), jnp.float32)
                            for _ in range(NUM_OFF)],
        ),
        compiler_params=pltpu.CompilerParams(
            dimension_semantics=("parallel",)),
    )(idx, y0, slab)
    return out


# speedup vs baseline: 5.2722x; 1.1125x over previous
import jax
import jax.numpy as jnp
from jax.experimental import pallas as pl
from jax.experimental.pallas import tpu as pltpu

T, DY, H = 8192, 256, 1024
R_W1Y, R_W2, R_B2, R_W3, R_B3 = 8192, 8448, 9472, 9480, 10504
DT = 100.0
SLOTS = (0, 0, 0, 0, 1, 1, 1, 1, 1, 1, 1, 1, 1, 2, 2, 2)
NUM_OFF = 3
TILE_B = 512
N_STEPS = 16


def _euler_kernel(idx_sref, y0_ref, slab_ref, out_ref, g0, g1, g2):
    i = pl.program_id(0)
    g = (g0, g1, g2)

    for mi in range(TILE_B):
        b = idx_sref[i * TILE_B + mi]
        for o in range(NUM_OFF):
            r = jnp.minimum(b + o, T - 1) if o else b
            c8 = pl.multiple_of((r >> 3) << 3, 8)
            chunk = slab_ref[pl.ds(c8, 8), :]
            row = pltpu.roll(chunk, -(r & 7), axis=0)[0:1, :]
            g[o][mi:mi + 1, :] = row

    hx = [go[...] for go in g]

    w1y = slab_ref[R_W1Y:R_W1Y + DY, :]
    w2 = slab_ref[R_W2:R_W2 + H, :]
    b2 = slab_ref[R_B2:R_B2 + 1, :]
    w3 = slab_ref[R_W3:R_W3 + H, :DY]
    b3 = slab_ref[R_B3:R_B3 + 1, :DY]

    y = y0_ref[...]
    for k in range(N_STEPS):
        h1 = jnp.tanh(hx[SLOTS[k]]
                      + jnp.dot(y, w1y, preferred_element_type=jnp.float32))
        h2 = jnp.tanh(jnp.dot(h1, w2, preferred_element_type=jnp.float32) + b2)
        y = y + DT * (jnp.dot(h2, w3, preferred_element_type=jnp.float32) + b3)

    out_ref[...] = y


def kernel(y0, base_idx, slab):
    batch, dy = y0.shape
    assert dy == DY
    idx = base_idx.astype(jnp.int32)

    out = pl.pallas_call(
        _euler_kernel,
        out_shape=jax.ShapeDtypeStruct((batch, DY), jnp.float32),
        grid_spec=pltpu.PrefetchScalarGridSpec(
            num_scalar_prefetch=1,
            grid=(batch // TILE_B,),
            in_specs=[
                pl.BlockSpec((TILE_B, DY), lambda i, idxs: (i, 0)),
                pl.BlockSpec(slab.shape, lambda i, idxs: (0, 0)),
            ],
            out_specs=pl.BlockSpec((TILE_B, DY), lambda i, idxs: (i, 0)),
            scratch_shapes=[pltpu.VMEM((TILE_B, H), jnp.float32)
                            for _ in range(NUM_OFF)],
        ),
        compiler_params=pltpu.CompilerParams(
            dimension_semantics=("parallel",)),
    )(idx, y0, slab)
    return out
